# baseline (speedup 1.0000x reference)
import jax
import jax.numpy as jnp
from jax import lax
from jax.experimental import pallas as pl
from jax.experimental.pallas import tpu as pltpu
from jax.experimental.pallas import tpu_sc as plsc

B = 4096
L = 50
D = 32
NC = 2
NS = 16
NW = NC * NS
ROWS_PW = B // NW
CB = 16
NCHUNK = ROWS_PW // CB
N = CB * L
LANES = 16


def _body(words_hbm, hp_hbm, hn_hbm, mask_hbm, cwv_hbm,
          pos_out, neg_out,
          words_v, hp_v, hn_v, mask_v,
          idxw, qp_v, qn_v, maskf,
          cr,
          scp, scn, outp_v, outn_v, sem):
    cid = lax.axis_index("c")
    sid = lax.axis_index("s")
    wid = sid * NC + cid
    iota = lax.iota(jnp.int32, LANES)

    @pl.loop(0, NCHUNK)
    def _chunk(c):
        pos0 = wid * (ROWS_PW * L) + c * N
        row0 = wid * ROWS_PW + c * CB

        pltpu.sync_copy(words_hbm.at[pl.ds(pos0, N)], words_v)
        pltpu.sync_copy(hp_hbm.at[pl.ds(pos0, N)], hp_v)
        pltpu.sync_copy(hn_hbm.at[pl.ds(pos0, N)], hn_v)
        pltpu.sync_copy(mask_hbm.at[pl.ds(pos0, N)], mask_v)

        @pl.loop(0, N // LANES)
        def _pha(g):
            sl = pl.ds(g * LANES, LANES)
            p = g * LANES + iota
            l = p % L
            rowbase = p - l
            m = mask_v[sl] != 0
            root = m | (l == 0)
            idxw[sl] = jnp.where(m, 0, words_v[sl])
            qp_v[sl] = rowbase + jnp.where(root, 0, hp_v[sl])
            qn_v[sl] = rowbase + jnp.where(root, 0, hn_v[sl])
            maskf[sl] = jnp.where(root, 0.0, 1.0)

        pltpu.async_copy(cwv_hbm.at[idxw], cr, sem)
        pltpu.make_async_copy(cwv_hbm.at[idxw], cr, sem).wait()

        @pl.loop(0, N // LANES)
        def _phc(g):
            sl = pl.ds(g * LANES, LANES)
            p16 = g * LANES + iota
            qp16 = qp_v[sl]
            qn16 = qn_v[sl]
            accp = jnp.zeros((LANES,), jnp.float32)
            accn = jnp.zeros((LANES,), jnp.float32)
            for d in range(D):
                dv = jnp.full((LANES,), d, jnp.int32)
                dv2 = jnp.full((LANES,), D + 1 + d, jnp.int32)
                wv = plsc.load_gather(cr, [p16, dv])
                accp = accp + wv * plsc.load_gather(cr, [qp16, dv2])
                accn = accn + wv * plsc.load_gather(cr, [qn16, dv2])
            m = maskf[sl]
            wbcol = jnp.full((LANES,), D, jnp.int32)
            vbcol = jnp.full((LANES,), 2 * D + 1, jnp.int32)
            wb16 = plsc.load_gather(cr, [p16, wbcol])
            scp[sl] = (accp + plsc.load_gather(cr, [qp16, vbcol]) + wb16) * m
            scn[sl] = (accn + plsc.load_gather(cr, [qn16, vbcol]) + wb16) * m

        accp = jnp.zeros((LANES,), jnp.float32)
        accn = jnp.zeros((LANES,), jnp.float32)
        rbase = iota * L
        for l in range(L):
            accp = accp + plsc.load_gather(scp, [rbase + l])
            accn = accn + plsc.load_gather(scn, [rbase + l])
        outp_v[...] = accp
        outn_v[...] = accn
        pltpu.sync_copy(outp_v, pos_out.at[pl.ds(row0, CB)])
        pltpu.sync_copy(outn_v, neg_out.at[pl.ds(row0, CB)])


def kernel(batch_id, words, head_ids, negative_head_ids, mask, V, W, vb, wb):
    del batch_id
    words_f = words.reshape(-1).astype(jnp.int32)
    hp_f = head_ids.reshape(-1).astype(jnp.int32)
    hn_f = negative_head_ids.reshape(-1).astype(jnp.int32)
    mask_f = mask.reshape(-1).astype(jnp.int32)
    pad = jnp.zeros((W.shape[0], 80 - 2 * (D + 1)), jnp.float32)
    cwv = jnp.concatenate([W, wb[:, None], V, vb[:, None], pad], axis=1)

    mesh = plsc.VectorSubcoreMesh(core_axis_name="c", subcore_axis_name="s")
    f = pl.kernel(
        _body,
        out_type=(
            jax.ShapeDtypeStruct((B,), jnp.float32),
            jax.ShapeDtypeStruct((B,), jnp.float32),
        ),
        mesh=mesh,
        compiler_params=pltpu.CompilerParams(needs_layout_passes=False,
                                             use_tc_tiling_on_sc=False),
        scratch_types=[
            pltpu.VMEM((N,), jnp.int32),
            pltpu.VMEM((N,), jnp.int32),
            pltpu.VMEM((N,), jnp.int32),
            pltpu.VMEM((N,), jnp.int32),
            pltpu.VMEM((N,), jnp.int32),
            pltpu.VMEM((N,), jnp.int32),
            pltpu.VMEM((N,), jnp.int32),
            pltpu.VMEM((N,), jnp.float32),
            pltpu.VMEM((N, 80), jnp.float32),
            pltpu.VMEM((N,), jnp.float32),
            pltpu.VMEM((N,), jnp.float32),
            pltpu.VMEM((LANES,), jnp.float32),
            pltpu.VMEM((LANES,), jnp.float32),
            pltpu.SemaphoreType.DMA,
        ],
    )
    return f(words_f, hp_f, hn_f, mask_f, cwv)


# speedup vs baseline: 3.7779x; 3.7779x over previous
import jax
import jax.numpy as jnp
from jax import lax
from jax.experimental import pallas as pl
from jax.experimental.pallas import tpu as pltpu
from jax.experimental.pallas import tpu_sc as plsc

B = 4096
L = 50
D = 32
NC = 2
NS = 16
NW = NC * NS
ROWS_PW = B // NW
CB = 16
NCHUNK = ROWS_PW // CB
N = CB * L
NG = N // 16
SUBC = 80
NSUB = N // SUBC
NSLOT = N
LANES = 16


def _body(words_hbm, hp_hbm, hn_hbm, mask_hbm, v_hbm, w_hbm, vb_hbm, wb_hbm,
          pos_out, neg_out,
          words_v, hp_v, hn_v, mask_v,
          cidx, cpos_v, qp_v, qn_v, maskf,
          wr, vr, wbv, vbv,
          scp, scn, outp_v, outn_v, sem):
    cid = lax.axis_index("c")
    sid = lax.axis_index("s")
    wid = sid * NC + cid
    iota = lax.iota(jnp.int32, LANES)
    zero16 = jnp.zeros((LANES,), jnp.float32)

    pltpu.sync_copy(v_hbm.at[pl.ds(0, 1)], vr.at[pl.ds(NSLOT, 1)])
    pltpu.sync_copy(vb_hbm.at[pl.ds(0, 8)], vbv.at[pl.ds(NSLOT, 8)])
    wr[0, pl.ds(0, LANES)] = zero16
    wr[0, pl.ds(LANES, LANES)] = zero16
    wbv[pl.ds(0, LANES)] = zero16

    @pl.loop(0, NCHUNK)
    def _chunk(c):
        pos0 = wid * (ROWS_PW * L) + c * N
        row0 = wid * ROWS_PW + c * CB

        pltpu.sync_copy(words_hbm.at[pl.ds(pos0, N)], words_v)
        pltpu.sync_copy(hp_hbm.at[pl.ds(pos0, N)], hp_v)
        pltpu.sync_copy(hn_hbm.at[pl.ds(pos0, N)], hn_v)
        pltpu.sync_copy(mask_hbm.at[pl.ds(pos0, N)], mask_v)

        @pl.loop(0, NG)
        def _phz(g):
            cidx[pl.ds(g * LANES, LANES)] = jnp.zeros((LANES,), jnp.int32)

        @pl.loop(0, NG, init_carry=jnp.int32(0))
        def _pha(g, k):
            sl = pl.ds(g * LANES, LANES)
            p = g * LANES + iota
            l = p % L
            m = mask_v[sl] != 0
            keep = jnp.logical_not(m)
            ki = jnp.where(keep, 1, 0)
            csum = plsc.cumsum(ki)
            cp = k + csum - ki
            cpos_v[sl] = cp
            plsc.store_scatter(cidx, [cp], words_v[sl], mask=keep)
            maskf[sl] = jnp.where(m | (l == 0), 0.0, 1.0)
            return k + jnp.sum(ki)

        k_tot = _pha

        @pl.loop(0, NG)
        def _pha2(g):
            sl = pl.ds(g * LANES, LANES)
            p = g * LANES + iota
            l = p % L
            rowbase = p - l
            root = (mask_v[sl] != 0) | (l == 0)
            qp = rowbase + jnp.where(root, 0, hp_v[sl])
            qn = rowbase + jnp.where(root, 0, hn_v[sl])
            tp = plsc.load_gather(mask_v, [qp]) != 0
            tn = plsc.load_gather(mask_v, [qn]) != 0
            qp_v[sl] = jnp.where(tp, NSLOT, plsc.load_gather(cpos_v, [qp]))
            qn_v[sl] = jnp.where(tn, NSLOT, plsc.load_gather(cpos_v, [qn]))

        for t in range(NSUB):
            @pl.when(t * SUBC < k_tot)
            def _():
                sl = pl.ds(t * SUBC, SUBC)
                pltpu.async_copy(w_hbm.at[cidx.at[sl]], wr.at[sl], sem)
                pltpu.async_copy(v_hbm.at[cidx.at[sl]], vr.at[sl], sem)
                pltpu.async_copy(wb_hbm.at[cidx.at[sl]], wbv.at[sl], sem)
                pltpu.async_copy(vb_hbm.at[cidx.at[sl]], vbv.at[sl], sem)
        for t in range(NSUB):
            @pl.when(t * SUBC < k_tot)
            def _():
                sl = pl.ds(t * SUBC, SUBC)
                pltpu.make_async_copy(w_hbm.at[cidx.at[sl]], wr.at[sl], sem).wait()
                pltpu.make_async_copy(v_hbm.at[cidx.at[sl]], vr.at[sl], sem).wait()
                pltpu.make_async_copy(wb_hbm.at[cidx.at[sl]], wbv.at[sl], sem).wait()
                pltpu.make_async_copy(vb_hbm.at[cidx.at[sl]], vbv.at[sl], sem).wait()

        @pl.loop(0, NG)
        def _phc(g):
            sl = pl.ds(g * LANES, LANES)
            p = g * LANES + iota
            l = p % L
            root = (mask_v[sl] != 0) | (l == 0)
            cw16 = jnp.where(root, 0, cpos_v[sl])
            qp16 = qp_v[sl]
            qn16 = qn_v[sl]
            accp = jnp.zeros((LANES,), jnp.float32)
            accn = jnp.zeros((LANES,), jnp.float32)
            for d in range(D):
                dv = jnp.full((LANES,), d, jnp.int32)
                wv = plsc.load_gather(wr, [cw16, dv])
                accp = accp + wv * plsc.load_gather(vr, [qp16, dv])
                accn = accn + wv * plsc.load_gather(vr, [qn16, dv])
            m = maskf[sl]
            wb16 = plsc.load_gather(wbv, [cw16])
            scp[sl] = (accp + plsc.load_gather(vbv, [qp16]) + wb16) * m
            scn[sl] = (accn + plsc.load_gather(vbv, [qn16]) + wb16) * m

        accp = jnp.zeros((LANES,), jnp.float32)
        accn = jnp.zeros((LANES,), jnp.float32)
        rbase = iota * L
        for l in range(L):
            accp = accp + plsc.load_gather(scp, [rbase + l])
            accn = accn + plsc.load_gather(scn, [rbase + l])
        outp_v[...] = accp
        outn_v[...] = accn
        pltpu.sync_copy(outp_v, pos_out.at[pl.ds(row0, CB)])
        pltpu.sync_copy(outn_v, neg_out.at[pl.ds(row0, CB)])


def kernel(batch_id, words, head_ids, negative_head_ids, mask, V, W, vb, wb):
    del batch_id
    words_f = words.reshape(-1).astype(jnp.int32)
    hp_f = head_ids.reshape(-1).astype(jnp.int32)
    hn_f = negative_head_ids.reshape(-1).astype(jnp.int32)
    mask_f = mask.reshape(-1).astype(jnp.int32)

    mesh = plsc.VectorSubcoreMesh(core_axis_name="c", subcore_axis_name="s")
    f = pl.kernel(
        _body,
        out_type=(
            jax.ShapeDtypeStruct((B,), jnp.float32),
            jax.ShapeDtypeStruct((B,), jnp.float32),
        ),
        mesh=mesh,
        compiler_params=pltpu.CompilerParams(needs_layout_passes=False,
                                             use_tc_tiling_on_sc=False),
        scratch_types=[
            pltpu.VMEM((N,), jnp.int32),
            pltpu.VMEM((N,), jnp.int32),
            pltpu.VMEM((N,), jnp.int32),
            pltpu.VMEM((N,), jnp.int32),
            pltpu.VMEM((N,), jnp.int32),
            pltpu.VMEM((N,), jnp.int32),
            pltpu.VMEM((N,), jnp.int32),
            pltpu.VMEM((N,), jnp.int32),
            pltpu.VMEM((N,), jnp.float32),
            pltpu.VMEM((N, D), jnp.float32),
            pltpu.VMEM((N + LANES, D), jnp.float32),
            pltpu.VMEM((N,), jnp.float32),
            pltpu.VMEM((N + LANES,), jnp.float32),
            pltpu.VMEM((N,), jnp.float32),
            pltpu.VMEM((N,), jnp.float32),
            pltpu.VMEM((LANES,), jnp.float32),
            pltpu.VMEM((LANES,), jnp.float32),
            pltpu.SemaphoreType.DMA,
        ],
    )
    return f(words_f, hp_f, hn_f, mask_f, V, W, vb, wb)


# speedup vs baseline: 3.8050x; 1.0072x over previous
import jax
import jax.numpy as jnp
from jax import lax
from jax.experimental import pallas as pl
from jax.experimental.pallas import tpu as pltpu
from jax.experimental.pallas import tpu_sc as plsc

B = 4096
L = 50
D = 32
NC = 2
NS = 16
NW = NC * NS
ROWS_PW = B // NW
NPW = ROWS_PW * L
CB = 16
NCHUNK = ROWS_PW // CB
N = CB * L
NG = N // 16
SUBC = 80
NSUB = N // SUBC
NSLOT = N
LANES = 16


def _body(words_hbm, hp_hbm, hn_hbm, mask_hbm, v_hbm, w_hbm, vb_hbm, wb_hbm,
          pos_out, neg_out,
          words_v, hp_v, hn_v, mask_v,
          cidx, cpos_v,
          wr, vr, wbv, vbv,
          outp_v, outn_v, sem):
    cid = lax.axis_index("c")
    sid = lax.axis_index("s")
    wid = sid * NC + cid
    iota = lax.iota(jnp.int32, LANES)
    rbase = iota * L
    zero16 = jnp.zeros((LANES,), jnp.float32)

    pltpu.sync_copy(v_hbm.at[pl.ds(0, 1)], vr.at[pl.ds(NSLOT, 1)])
    pltpu.sync_copy(vb_hbm.at[pl.ds(0, 8)], vbv.at[pl.ds(NSLOT, 8)])
    wr[0, pl.ds(0, LANES)] = zero16
    wr[0, pl.ds(LANES, LANES)] = zero16
    wbv[pl.ds(0, LANES)] = zero16

    @pl.loop(0, NG)
    def _phz(g):
        cidx[pl.ds(g * LANES, LANES)] = jnp.zeros((LANES,), jnp.int32)

    base = wid * NPW
    pltpu.async_copy(words_hbm.at[pl.ds(base, NPW)], words_v, sem)
    pltpu.async_copy(hp_hbm.at[pl.ds(base, NPW)], hp_v, sem)
    pltpu.async_copy(hn_hbm.at[pl.ds(base, NPW)], hn_v, sem)
    pltpu.async_copy(mask_hbm.at[pl.ds(base, NPW)], mask_v, sem)
    pltpu.make_async_copy(words_hbm.at[pl.ds(base, NPW)], words_v, sem).wait()
    pltpu.make_async_copy(hp_hbm.at[pl.ds(base, NPW)], hp_v, sem).wait()
    pltpu.make_async_copy(hn_hbm.at[pl.ds(base, NPW)], hn_v, sem).wait()
    pltpu.make_async_copy(mask_hbm.at[pl.ds(base, NPW)], mask_v, sem).wait()

    @pl.loop(0, NCHUNK)
    def _chunk(c):
        p0 = c * N
        row0 = wid * ROWS_PW + c * CB

        @pl.loop(0, L, init_carry=jnp.int32(0))
        def _pha(l, k):
            p = p0 + rbase + l
            m = plsc.load_gather(mask_v, [p]) != 0
            keep = jnp.logical_not(m)
            ki = jnp.where(keep, 1, 0)
            csum = plsc.cumsum(ki)
            cp = k + csum - ki
            plsc.store_scatter(cpos_v, [rbase + l], cp)
            w16 = plsc.load_gather(words_v, [p])
            plsc.store_scatter(cidx, [cp], w16, mask=keep)
            return k + jnp.sum(ki)

        k_tot = _pha

        for t in range(NSUB):
            @pl.when(t * SUBC < k_tot)
            def _():
                sl = pl.ds(t * SUBC, SUBC)
                pltpu.async_copy(w_hbm.at[cidx.at[sl]], wr.at[sl], sem)
                pltpu.async_copy(v_hbm.at[cidx.at[sl]], vr.at[sl], sem)
                pltpu.async_copy(wb_hbm.at[cidx.at[sl]], wbv.at[sl], sem)
                pltpu.async_copy(vb_hbm.at[cidx.at[sl]], vbv.at[sl], sem)
        for t in range(NSUB):
            @pl.when(t * SUBC < k_tot)
            def _():
                sl = pl.ds(t * SUBC, SUBC)
                pltpu.make_async_copy(w_hbm.at[cidx.at[sl]], wr.at[sl], sem).wait()
                pltpu.make_async_copy(v_hbm.at[cidx.at[sl]], vr.at[sl], sem).wait()
                pltpu.make_async_copy(wb_hbm.at[cidx.at[sl]], wbv.at[sl], sem).wait()
                pltpu.make_async_copy(vb_hbm.at[cidx.at[sl]], vbv.at[sl], sem).wait()

        outp_v[...] = zero16
        outn_v[...] = zero16

        @pl.loop(0, L)
        def _phc(l):
            pin = rbase + l
            p = p0 + pin
            m16 = plsc.load_gather(mask_v, [p]) != 0
            root = jnp.logical_or(m16, l == 0)
            cw16 = jnp.where(root, 0, plsc.load_gather(cpos_v, [pin]))
            hp16 = plsc.load_gather(hp_v, [p])
            hn16 = plsc.load_gather(hn_v, [p])
            qpp = rbase + jnp.where(root, 0, hp16)
            qnp = rbase + jnp.where(root, 0, hn16)
            tp = plsc.load_gather(mask_v, [p0 + qpp]) != 0
            tn = plsc.load_gather(mask_v, [p0 + qnp]) != 0
            qp16 = jnp.where(tp, NSLOT, plsc.load_gather(cpos_v, [qpp]))
            qn16 = jnp.where(tn, NSLOT, plsc.load_gather(cpos_v, [qnp]))
            accp = jnp.zeros((LANES,), jnp.float32)
            accn = jnp.zeros((LANES,), jnp.float32)
            for d in range(D):
                dv = jnp.full((LANES,), d, jnp.int32)
                wv = plsc.load_gather(wr, [cw16, dv])
                accp = accp + wv * plsc.load_gather(vr, [qp16, dv])
                accn = accn + wv * plsc.load_gather(vr, [qn16, dv])
            mf = jnp.where(root, 0.0, 1.0)
            wb16 = plsc.load_gather(wbv, [cw16])
            outp_v[...] = outp_v[...] + (accp + plsc.load_gather(vbv, [qp16]) + wb16) * mf
            outn_v[...] = outn_v[...] + (accn + plsc.load_gather(vbv, [qn16]) + wb16) * mf

        pltpu.sync_copy(outp_v, pos_out.at[pl.ds(row0, CB)])
        pltpu.sync_copy(outn_v, neg_out.at[pl.ds(row0, CB)])


def kernel(batch_id, words, head_ids, negative_head_ids, mask, V, W, vb, wb):
    del batch_id
    words_f = words.reshape(-1).astype(jnp.int32)
    hp_f = head_ids.reshape(-1).astype(jnp.int32)
    hn_f = negative_head_ids.reshape(-1).astype(jnp.int32)
    mask_f = mask.reshape(-1).astype(jnp.int32)

    mesh = plsc.VectorSubcoreMesh(core_axis_name="c", subcore_axis_name="s")
    f = pl.kernel(
        _body,
        out_type=(
            jax.ShapeDtypeStruct((B,), jnp.float32),
            jax.ShapeDtypeStruct((B,), jnp.float32),
        ),
        mesh=mesh,
        compiler_params=pltpu.CompilerParams(needs_layout_passes=False,
                                             use_tc_tiling_on_sc=False),
        scratch_types=[
            pltpu.VMEM((NPW,), jnp.int32),
            pltpu.VMEM((NPW,), jnp.int32),
            pltpu.VMEM((NPW,), jnp.int32),
            pltpu.VMEM((NPW,), jnp.int32),
            pltpu.VMEM((N,), jnp.int32),
            pltpu.VMEM((N,), jnp.int32),
            pltpu.VMEM((N, D), jnp.float32),
            pltpu.VMEM((N + LANES, D), jnp.float32),
            pltpu.VMEM((N,), jnp.float32),
            pltpu.VMEM((N + LANES,), jnp.float32),
            pltpu.VMEM((LANES,), jnp.float32),
            pltpu.VMEM((LANES,), jnp.float32),
            pltpu.SemaphoreType.DMA,
        ],
    )
    return f(words_f, hp_f, hn_f, mask_f, V, W, vb, wb)


# speedup vs baseline: 3.8434x; 1.0101x over previous
import jax
import jax.numpy as jnp
from jax import lax
from jax.experimental import pallas as pl
from jax.experimental.pallas import tpu as pltpu
from jax.experimental.pallas import tpu_sc as plsc

B = 4096
L = 50
D = 32
NC = 2
NS = 16
NW = NC * NS
ROWS_PW = B // NW
NPW = ROWS_PW * L
CB = 16
NCHUNK = ROWS_PW // CB
N = CB * L
NG = N // 16
SUBC = 80
NSUB = N // SUBC
NSLOT = N
LANES = 16


def _body(words_hbm, hp_hbm, hn_hbm, mask_hbm, v_hbm, w_hbm, vb_hbm, wb_hbm,
          pos_out, neg_out,
          words_v, hp_v, hn_v, mask_v,
          cidx, cpos_v, qp_v, qn_v, maskf,
          wr, vr, wbv, vbv,
          scp, scn, outp_v, outn_v, sem):
    cid = lax.axis_index("c")
    sid = lax.axis_index("s")
    wid = sid * NC + cid
    iota = lax.iota(jnp.int32, LANES)
    zero16 = jnp.zeros((LANES,), jnp.float32)

    pltpu.sync_copy(v_hbm.at[pl.ds(0, 1)], vr.at[pl.ds(NSLOT, 1)])
    pltpu.sync_copy(vb_hbm.at[pl.ds(0, 8)], vbv.at[pl.ds(NSLOT, 8)])
    wr[0, pl.ds(0, LANES)] = zero16
    wr[0, pl.ds(LANES, LANES)] = zero16
    wbv[pl.ds(0, LANES)] = zero16

    @pl.loop(0, NG)
    def _phz(g):
        cidx[pl.ds(g * LANES, LANES)] = jnp.zeros((LANES,), jnp.int32)

    base = wid * NPW
    pltpu.async_copy(words_hbm.at[pl.ds(base, NPW)], words_v, sem)
    pltpu.async_copy(hp_hbm.at[pl.ds(base, NPW)], hp_v, sem)
    pltpu.async_copy(hn_hbm.at[pl.ds(base, NPW)], hn_v, sem)
    pltpu.async_copy(mask_hbm.at[pl.ds(base, NPW)], mask_v, sem)
    pltpu.make_async_copy(words_hbm.at[pl.ds(base, NPW)], words_v, sem).wait()
    pltpu.make_async_copy(hp_hbm.at[pl.ds(base, NPW)], hp_v, sem).wait()
    pltpu.make_async_copy(hn_hbm.at[pl.ds(base, NPW)], hn_v, sem).wait()
    pltpu.make_async_copy(mask_hbm.at[pl.ds(base, NPW)], mask_v, sem).wait()

    @pl.loop(0, NCHUNK)
    def _chunk(c):
        p0 = c * N
        row0 = wid * ROWS_PW + c * CB

        @pl.loop(0, NG, init_carry=jnp.int32(0))
        def _pha(g, k):
            sl = pl.ds(g * LANES, LANES)
            sli = pl.ds(p0 + g * LANES, LANES)
            p = g * LANES + iota
            l = p % L
            m = mask_v[sli] != 0
            keep = jnp.logical_not(m)
            ki = jnp.where(keep, 1, 0)
            csum = plsc.cumsum(ki)
            cp = k + csum - ki
            cpos_v[sl] = cp
            plsc.store_scatter(cidx, [cp], words_v[sli], mask=keep)
            maskf[sl] = jnp.where(m | (l == 0), 0.0, 1.0)
            return k + jnp.sum(ki)

        k_tot = _pha

        @pl.loop(0, NG)
        def _pha2(g):
            sl = pl.ds(g * LANES, LANES)
            sli = pl.ds(p0 + g * LANES, LANES)
            p = g * LANES + iota
            l = p % L
            rowbase = p - l
            root = (mask_v[sli] != 0) | (l == 0)
            qp = rowbase + jnp.where(root, 0, hp_v[sli])
            qn = rowbase + jnp.where(root, 0, hn_v[sli])
            tp = plsc.load_gather(mask_v, [p0 + qp]) != 0
            tn = plsc.load_gather(mask_v, [p0 + qn]) != 0
            qp_v[sl] = jnp.where(tp, NSLOT, plsc.load_gather(cpos_v, [qp]))
            qn_v[sl] = jnp.where(tn, NSLOT, plsc.load_gather(cpos_v, [qn]))

        for t in range(NSUB):
            @pl.when(t * SUBC < k_tot)
            def _():
                sl = pl.ds(t * SUBC, SUBC)
                pltpu.async_copy(w_hbm.at[cidx.at[sl]], wr.at[sl], sem)
                pltpu.async_copy(v_hbm.at[cidx.at[sl]], vr.at[sl], sem)
                pltpu.async_copy(wb_hbm.at[cidx.at[sl]], wbv.at[sl], sem)
                pltpu.async_copy(vb_hbm.at[cidx.at[sl]], vbv.at[sl], sem)
        for t in range(NSUB):
            @pl.when(t * SUBC < k_tot)
            def _():
                sl = pl.ds(t * SUBC, SUBC)
                pltpu.make_async_copy(w_hbm.at[cidx.at[sl]], wr.at[sl], sem).wait()
                pltpu.make_async_copy(v_hbm.at[cidx.at[sl]], vr.at[sl], sem).wait()
                pltpu.make_async_copy(wb_hbm.at[cidx.at[sl]], wbv.at[sl], sem).wait()
                pltpu.make_async_copy(vb_hbm.at[cidx.at[sl]], vbv.at[sl], sem).wait()

        @pl.loop(0, NG)
        def _phc(g):
            sl = pl.ds(g * LANES, LANES)
            sli = pl.ds(p0 + g * LANES, LANES)
            p = g * LANES + iota
            l = p % L
            root = (mask_v[sli] != 0) | (l == 0)
            cw16 = jnp.where(root, 0, cpos_v[sl])
            qp16 = qp_v[sl]
            qn16 = qn_v[sl]
            accp = jnp.zeros((LANES,), jnp.float32)
            accn = jnp.zeros((LANES,), jnp.float32)
            for d in range(D):
                dv = jnp.full((LANES,), d, jnp.int32)
                wv = plsc.load_gather(wr, [cw16, dv])
                accp = accp + wv * plsc.load_gather(vr, [qp16, dv])
                accn = accn + wv * plsc.load_gather(vr, [qn16, dv])
            m = maskf[sl]
            wb16 = plsc.load_gather(wbv, [cw16])
            scp[sl] = (accp + plsc.load_gather(vbv, [qp16]) + wb16) * m
            scn[sl] = (accn + plsc.load_gather(vbv, [qn16]) + wb16) * m

        accp = jnp.zeros((LANES,), jnp.float32)
        accn = jnp.zeros((LANES,), jnp.float32)
        rbase = iota * L
        for l in range(L):
            accp = accp + plsc.load_gather(scp, [rbase + l])
            accn = accn + plsc.load_gather(scn, [rbase + l])
        outp_v[...] = accp
        outn_v[...] = accn
        pltpu.sync_copy(outp_v, pos_out.at[pl.ds(row0, CB)])
        pltpu.sync_copy(outn_v, neg_out.at[pl.ds(row0, CB)])


def kernel(batch_id, words, head_ids, negative_head_ids, mask, V, W, vb, wb):
    del batch_id
    words_f = words.reshape(-1).astype(jnp.int32)
    hp_f = head_ids.reshape(-1).astype(jnp.int32)
    hn_f = negative_head_ids.reshape(-1).astype(jnp.int32)
    mask_f = mask.reshape(-1).astype(jnp.int32)

    mesh = plsc.VectorSubcoreMesh(core_axis_name="c", subcore_axis_name="s")
    f = pl.kernel(
        _body,
        out_type=(
            jax.ShapeDtypeStruct((B,), jnp.float32),
            jax.ShapeDtypeStruct((B,), jnp.float32),
        ),
        mesh=mesh,
        compiler_params=pltpu.CompilerParams(needs_layout_passes=False,
                                             use_tc_tiling_on_sc=False),
        scratch_types=[
            pltpu.VMEM((NPW,), jnp.int32),
            pltpu.VMEM((NPW,), jnp.int32),
            pltpu.VMEM((NPW,), jnp.int32),
            pltpu.VMEM((NPW,), jnp.int32),
            pltpu.VMEM((N,), jnp.int32),
            pltpu.VMEM((N,), jnp.int32),
            pltpu.VMEM((N,), jnp.int32),
            pltpu.VMEM((N,), jnp.int32),
            pltpu.VMEM((N,), jnp.float32),
            pltpu.VMEM((N, D), jnp.float32),
            pltpu.VMEM((N + LANES, D), jnp.float32),
            pltpu.VMEM((N,), jnp.float32),
            pltpu.VMEM((N + LANES,), jnp.float32),
            pltpu.VMEM((N,), jnp.float32),
            pltpu.VMEM((N,), jnp.float32),
            pltpu.VMEM((LANES,), jnp.float32),
            pltpu.VMEM((LANES,), jnp.float32),
            pltpu.SemaphoreType.DMA,
        ],
    )
    return f(words_f, hp_f, hn_f, mask_f, V, W, vb, wb)
